# baseline (speedup 1.0000x reference)
import functools

import jax
import jax.numpy as jnp
from jax import lax
from jax.experimental import pallas as pl
from jax.experimental.pallas import tpu as pltpu
from jax.experimental.pallas import tpu_sc as plsc

S = 200
D = 64
B = 1024
NC = 2
NS = 16
NW = NC * NS
SEQ_PER_W = B // NW
SPLIT0 = 128
SPLIT1 = S - SPLIT0


def _body(x_hbm, po_hbm, tab_hbm, pos_hbm, out_hbm,
          idx_v, po_v, stag_v, pos_v, gsem, osem):
    wid = lax.axis_index("s") * NC + lax.axis_index("c")
    base = wid * SEQ_PER_W
    pltpu.sync_copy(pos_hbm, pos_v)
    pltpu.sync_copy(x_hbm.at[pl.ds(base * S, SEQ_PER_W * S)], idx_v)
    pltpu.sync_copy(po_hbm.at[pl.ds(base * S, SEQ_PER_W * S)], po_v)

    def fire(i, b):
        off = i * S
        pltpu.async_copy(
            tab_hbm.at[idx_v.at[pl.ds(off, SPLIT0)]],
            stag_v.at[b].at[pl.ds(0, SPLIT0)], gsem)
        pltpu.async_copy(
            tab_hbm.at[idx_v.at[pl.ds(off + SPLIT0, SPLIT1)]],
            stag_v.at[b].at[pl.ds(SPLIT0, SPLIT1)], gsem)

    def drain_gathers(i, b):
        off = i * S
        pltpu.make_async_copy(
            tab_hbm.at[idx_v.at[pl.ds(off, SPLIT0)]],
            stag_v.at[b].at[pl.ds(0, SPLIT0)], gsem).wait()
        pltpu.make_async_copy(
            tab_hbm.at[idx_v.at[pl.ds(off + SPLIT0, SPLIT1)]],
            stag_v.at[b].at[pl.ds(SPLIT0, SPLIT1)], gsem).wait()

    fire(0, 0)

    def seq_body(i, carry):
        b = lax.rem(i, 2)
        drain_gathers(i, b)

        @pl.when(i >= 1)
        def _():
            pltpu.make_async_copy(
                stag_v.at[1 - b],
                out_hbm.at[base + i - 1], osem).wait()

        @pl.when(i + 1 < SEQ_PER_W)
        def _():
            fire(i + 1, 1 - b)

        off = i * S

        def row_body(r, c2):
            p = po_v[pl.ds(off + r, 16)][0]
            for c in range(D // 16):
                sl = pl.ds(c * 16, 16)
                stag_v[b, r, sl] = stag_v[b, r, pl.ds(p + c * 16, 16)] \
                    + pos_v[r, sl]
            return c2

        lax.fori_loop(0, S, row_body, 0)
        pltpu.async_copy(
            stag_v.at[b], out_hbm.at[base + i], osem)
        return carry

    lax.fori_loop(0, SEQ_PER_W, seq_body, 0)
    pltpu.make_async_copy(
        stag_v.at[(SEQ_PER_W - 1) % 2],
        out_hbm.at[base + SEQ_PER_W - 1], osem).wait()


@functools.partial(
    pl.kernel,
    mesh=plsc.VectorSubcoreMesh(core_axis_name="c", subcore_axis_name="s"),
    compiler_params=pltpu.CompilerParams(use_tc_tiling_on_sc=True),
    out_type=jax.ShapeDtypeStruct((B, S, 2 * D), jnp.float32),
    scratch_types=[
        pltpu.VMEM((SEQ_PER_W * S,), jnp.int32),
        pltpu.VMEM((SEQ_PER_W * S,), jnp.int32),
        pltpu.VMEM((2, S, 2 * D), jnp.float32),
        pltpu.VMEM((S, D), jnp.float32),
        pltpu.SemaphoreType.DMA,
        pltpu.SemaphoreType.DMA,
    ],
)
def _embed(x_hbm, po_hbm, tab_hbm, pos_hbm, out_hbm,
           idx_v, po_v, stag_v, pos_v, gsem, osem):
    _body(x_hbm, po_hbm, tab_hbm, pos_hbm, out_hbm,
          idx_v, po_v, stag_v, pos_v, gsem, osem)


@jax.jit
def kernel(x, token_table, pos_table):
    xf = x.reshape(B * S)
    tab2 = token_table.reshape(500000, 2 * D)
    return _embed(xf >> 1, (xf & 1) * D, tab2, pos_table)[:, :, :D]


# speedup vs baseline: 1.2087x; 1.2087x over previous
import functools

import jax
import jax.numpy as jnp
from jax import lax
from jax.experimental import pallas as pl
from jax.experimental.pallas import tpu as pltpu
from jax.experimental.pallas import tpu_sc as plsc

S = 200
D = 64
B = 1024
NC = 2
NS = 16
NW = NC * NS
SEQ_PER_W = B // NW
SPLIT0 = 128
SPLIT1 = S - SPLIT0


def _body(x_hbm, tab_hbm, pos_hbm, out_hbm,
          idx_v, stag_v, pos_v, gsem, osem):
    wid = lax.axis_index("s") * NC + lax.axis_index("c")
    base = wid * SEQ_PER_W
    pltpu.sync_copy(pos_hbm, pos_v)
    pltpu.sync_copy(x_hbm.at[pl.ds(base * S, SEQ_PER_W * S)], idx_v)

    def fire(i, b):
        off = i * S
        pltpu.async_copy(
            tab_hbm.at[idx_v.at[pl.ds(off, SPLIT0)]],
            stag_v.at[b].at[pl.ds(0, SPLIT0)], gsem)
        pltpu.async_copy(
            tab_hbm.at[idx_v.at[pl.ds(off + SPLIT0, SPLIT1)]],
            stag_v.at[b].at[pl.ds(SPLIT0, SPLIT1)], gsem)

    def drain_gathers(i, b):
        off = i * S
        pltpu.make_async_copy(
            tab_hbm.at[idx_v.at[pl.ds(off, SPLIT0)]],
            stag_v.at[b].at[pl.ds(0, SPLIT0)], gsem).wait()
        pltpu.make_async_copy(
            tab_hbm.at[idx_v.at[pl.ds(off + SPLIT0, SPLIT1)]],
            stag_v.at[b].at[pl.ds(SPLIT0, SPLIT1)], gsem).wait()

    fire(0, 0)

    def seq_body(i, carry):
        b = lax.rem(i, 2)
        drain_gathers(i, b)

        @pl.when(i >= 1)
        def _():
            pltpu.make_async_copy(
                stag_v.at[1 - b],
                out_hbm.at[base + i - 1], osem).wait()

        @pl.when(i + 1 < SEQ_PER_W)
        def _():
            fire(i + 1, 1 - b)

        def row_body(r8, c2):
            for dr in range(8):
                r = r8 * 8 + dr
                for c in range(D // 16):
                    sl = pl.ds(c * 16, 16)
                    stag_v[b, r, sl] = stag_v[b, r, sl] + pos_v[r, sl]
            return c2

        lax.fori_loop(0, S // 8, row_body, 0)
        pltpu.async_copy(
            stag_v.at[b], out_hbm.at[base + i], osem)
        return carry

    lax.fori_loop(0, SEQ_PER_W, seq_body, 0)
    pltpu.make_async_copy(
        stag_v.at[(SEQ_PER_W - 1) % 2],
        out_hbm.at[base + SEQ_PER_W - 1], osem).wait()


@functools.partial(
    pl.kernel,
    mesh=plsc.VectorSubcoreMesh(core_axis_name="c", subcore_axis_name="s"),
    compiler_params=pltpu.CompilerParams(use_tc_tiling_on_sc=True),
    out_type=jax.ShapeDtypeStruct((B, S, 2 * D), jnp.float32),
    scratch_types=[
        pltpu.VMEM((SEQ_PER_W * S,), jnp.int32),
        pltpu.VMEM((2, S, 2 * D), jnp.float32),
        pltpu.VMEM((S, D), jnp.float32),
        pltpu.SemaphoreType.DMA,
        pltpu.SemaphoreType.DMA,
    ],
)
def _embed(x_hbm, tab_hbm, pos_hbm, out_hbm, idx_v, stag_v, pos_v, gsem, osem):
    _body(x_hbm, tab_hbm, pos_hbm, out_hbm, idx_v, stag_v, pos_v, gsem, osem)


@jax.jit
def kernel(x, token_table, pos_table):
    tabp = jnp.pad(token_table, ((0, 0), (0, D)))
    return _embed(x.reshape(B * S), tabp, pos_table)[:, :, :D]
